# baseline (speedup 1.0000x reference)
import functools

import jax
import jax.numpy as jnp
from jax import lax
from jax.experimental import pallas as pl
from jax.experimental.pallas import tpu as pltpu
from jax.experimental.pallas import tpu_sc as plsc

N_NODES = 10000
D_FEAT = 128
HIDDEN = 128
N_CLASSES = 64
N_EDGES = 320000

NC = 2
NS = 16
NW = NC * NS
CHUNK = 128
N_HALF = 2
CHUNKS_PER_HALF = 40
CHUNKS_PER_TILE = N_HALF * CHUNKS_PER_HALF
E_PAD = NW * CHUNKS_PER_TILE * CHUNK
ROWS_PER_TILE = 640
N_ACC = NS * ROWS_PER_TILE
DUMMY_ROW = N_NODES


def _seg_sum_body(table, src_idx, dst_idx, zeros, out,
                  src_v, dst_v, rows_a, rows_b, acc,
                  sem_a, sem_b, ssem_a, ssem_b):
    c = lax.axis_index("c")
    s = lax.axis_index("s")
    wid = s * NC + c

    pltpu.sync_copy(zeros.at[pl.ds(s * ROWS_PER_TILE, ROWS_PER_TILE)],
                    acc.at[pl.ds(s * ROWS_PER_TILE, ROWS_PER_TILE)])
    plsc.subcore_barrier()

    for h in range(N_HALF):
        pltpu.sync_copy(src_idx.at[wid, h], src_v)
        pltpu.sync_copy(dst_idx.at[wid, h], dst_v)
        pltpu.async_copy(table.at[src_v.at[0]], rows_a, sem_a)

        def step(g, carry):
            j = 2 * g
            pltpu.async_copy(table.at[src_v.at[j + 1]], rows_b, sem_b)
            pltpu.make_async_copy(table.at[src_v.at[j]], rows_a, sem_a).wait()
            pltpu.async_copy(rows_a, acc.at[dst_v.at[j]], ssem_a, add=True)
            pltpu.make_async_copy(table.at[src_v.at[j + 1]], rows_b, sem_b).wait()
            pltpu.async_copy(rows_b, acc.at[dst_v.at[j + 1]], ssem_b, add=True)
            pltpu.make_async_copy(rows_a, acc.at[dst_v.at[j]], ssem_a).wait()

            @pl.when(j + 2 < CHUNKS_PER_HALF)
            def _():
                pltpu.async_copy(table.at[src_v.at[j + 2]], rows_a, sem_a)

            pltpu.make_async_copy(rows_b, acc.at[dst_v.at[j + 1]], ssem_b).wait()
            return carry

        lax.fori_loop(0, CHUNKS_PER_HALF // 2, step, 0)

    plsc.subcore_barrier()

    def wstep(k, carry):
        off = s * ROWS_PER_TILE + k * CHUNK
        pltpu.sync_copy(acc.at[pl.ds(off, CHUNK)], rows_a)
        pltpu.sync_copy(rows_a, out.at[c, pl.ds(off, CHUNK)])
        return carry

    lax.fori_loop(0, ROWS_PER_TILE // CHUNK, wstep, 0)


_seg_sum = pl.kernel(
    _seg_sum_body,
    out_type=jax.ShapeDtypeStruct((NC, N_ACC, D_FEAT), jnp.float32),
    mesh=plsc.VectorSubcoreMesh(core_axis_name="c", subcore_axis_name="s"),
    scratch_types=[
        pltpu.VMEM((CHUNKS_PER_HALF, CHUNK), jnp.int32),
        pltpu.VMEM((CHUNKS_PER_HALF, CHUNK), jnp.int32),
        pltpu.VMEM((CHUNK, D_FEAT), jnp.float32),
        pltpu.VMEM((CHUNK, D_FEAT), jnp.float32),
        pltpu.VMEM_SHARED((N_ACC, D_FEAT), jnp.float32),
        pltpu.SemaphoreType.DMA,
        pltpu.SemaphoreType.DMA,
        pltpu.SemaphoreType.DMA,
        pltpu.SemaphoreType.DMA,
    ],
)


def _mlp1_body(x_ref, p_ref, w1a_ref, gamma_ref, beta_ref, mean_ref, var_ref,
               w1b_ref, h_ref):
    z = x_ref[...] + p_ref[0] + p_ref[1]
    t = jnp.dot(z, w1a_ref[...], preferred_element_type=jnp.float32)
    rs = lax.rsqrt(var_ref[...] + 1e-5)
    t = (t - mean_ref[...]) * rs * gamma_ref[...] + beta_ref[...]
    t = jnp.maximum(t, 0.0)
    t = jnp.dot(t, w1b_ref[...], preferred_element_type=jnp.float32)
    h_ref[...] = jnp.maximum(t, 0.0)


def _mlp2_body(h_ref, p_ref, w2a_ref, w2b_ref, o_ref):
    z = h_ref[...] + p_ref[0] + p_ref[1]
    t = jnp.dot(z, w2a_ref[...], preferred_element_type=jnp.float32)
    t = jnp.maximum(t, 0.0)
    t = jnp.dot(t, w2b_ref[...], preferred_element_type=jnp.float32)
    o_ref[...] = jnp.maximum(t, 0.0)


M_BLK = 1000
_GRID = (N_NODES // M_BLK,)


def _row_spec(d):
    return pl.BlockSpec((M_BLK, d), lambda i: (i, 0))


def _part_spec():
    return pl.BlockSpec((2, M_BLK, D_FEAT), lambda i: (0, i, 0))


def _full_spec(a, b):
    return pl.BlockSpec((a, b), lambda i: (0, 0))


_mlp1 = pl.pallas_call(
    _mlp1_body,
    grid=_GRID,
    in_specs=[
        _row_spec(D_FEAT),
        _part_spec(),
        _full_spec(D_FEAT, HIDDEN),
        _full_spec(1, HIDDEN),
        _full_spec(1, HIDDEN),
        _full_spec(1, HIDDEN),
        _full_spec(1, HIDDEN),
        _full_spec(HIDDEN, HIDDEN),
    ],
    out_specs=_row_spec(HIDDEN),
    out_shape=jax.ShapeDtypeStruct((N_NODES, HIDDEN), jnp.float32),
)

_mlp2 = pl.pallas_call(
    _mlp2_body,
    grid=_GRID,
    in_specs=[
        _row_spec(HIDDEN),
        _part_spec(),
        _full_spec(HIDDEN, HIDDEN),
        _full_spec(HIDDEN, N_CLASSES),
    ],
    out_specs=_row_spec(N_CLASSES),
    out_shape=jax.ShapeDtypeStruct((N_NODES, N_CLASSES), jnp.float32),
)


@jax.jit
def kernel(x, edge_index, W1a, bn_gamma, bn_beta, bn_mean, bn_var, W1b, W2a, W2b):
    ei = edge_index.astype(jnp.int32)
    pad = E_PAD - N_EDGES
    pad_dst = N_NODES + (jnp.arange(pad, dtype=jnp.int32) % (N_ACC - N_NODES))
    pad_src = jnp.arange(pad, dtype=jnp.int32) % N_NODES
    src = jnp.concatenate([ei[0], pad_src])
    dst = jnp.concatenate([ei[1], pad_dst])
    src3 = src.reshape(NW, N_HALF, CHUNKS_PER_HALF, CHUNK)
    dst3 = dst.reshape(NW, N_HALF, CHUNKS_PER_HALF, CHUNK)
    zeros = jnp.zeros((N_ACC, D_FEAT), jnp.float32)

    p1 = _seg_sum(x, src3, dst3, zeros)
    h = _mlp1(x, p1[:, :N_NODES], W1a,
              bn_gamma.reshape(1, HIDDEN), bn_beta.reshape(1, HIDDEN),
              bn_mean.reshape(1, HIDDEN), bn_var.reshape(1, HIDDEN), W1b)
    p2 = _seg_sum(h, src3, dst3, zeros)
    return _mlp2(h, p2[:, :N_NODES], W2a, W2b)


# speedup vs baseline: 1.3016x; 1.3016x over previous
import functools

import jax
import jax.numpy as jnp
from jax import lax
from jax.experimental import pallas as pl
from jax.experimental.pallas import tpu as pltpu
from jax.experimental.pallas import tpu_sc as plsc

N_NODES = 10000
D_FEAT = 128
HIDDEN = 128
N_CLASSES = 64
N_EDGES = 320000

NC = 2
NS = 16
NW = NC * NS
CHUNK = 128
N_HALF = 2
CHUNKS_PER_HALF = 40
CHUNKS_PER_TILE = N_HALF * CHUNKS_PER_HALF
E_PAD = NW * CHUNKS_PER_TILE * CHUNK
ROWS_PER_TILE = 640
N_ACC = NS * ROWS_PER_TILE
DUMMY_ROW = N_NODES


def _seg_sum_body(table, src_idx, dst_idx, zeros, out,
                  src_v, dst_v, rows_a, rows_b, acc, sem_a, sem_b):
    c = lax.axis_index("c")
    s = lax.axis_index("s")
    wid = s * NC + c

    pltpu.sync_copy(zeros.at[pl.ds(s * ROWS_PER_TILE, ROWS_PER_TILE)],
                    acc.at[pl.ds(s * ROWS_PER_TILE, ROWS_PER_TILE)])
    plsc.subcore_barrier()

    for h in range(N_HALF):
        pltpu.sync_copy(src_idx.at[wid, h], src_v)
        pltpu.sync_copy(dst_idx.at[wid, h], dst_v)
        pltpu.async_copy(table.at[src_v.at[0]], rows_a, sem_a)

        def step(g, carry):
            j = 2 * g
            pltpu.async_copy(table.at[src_v.at[j + 1]], rows_b, sem_b)
            pltpu.make_async_copy(table.at[src_v.at[j]], rows_a, sem_a).wait()
            pltpu.sync_copy(rows_a, acc.at[dst_v.at[j]], add=True)

            @pl.when(j + 2 < CHUNKS_PER_HALF)
            def _():
                pltpu.async_copy(table.at[src_v.at[j + 2]], rows_a, sem_a)

            pltpu.make_async_copy(table.at[src_v.at[j + 1]], rows_b, sem_b).wait()
            pltpu.sync_copy(rows_b, acc.at[dst_v.at[j + 1]], add=True)
            return carry

        lax.fori_loop(0, CHUNKS_PER_HALF // 2, step, 0)

    plsc.subcore_barrier()

    def wstep(k, carry):
        off = s * ROWS_PER_TILE + k * CHUNK
        pltpu.sync_copy(acc.at[pl.ds(off, CHUNK)], rows_a)
        pltpu.sync_copy(rows_a, out.at[c, pl.ds(off, CHUNK)])
        return carry

    lax.fori_loop(0, ROWS_PER_TILE // CHUNK, wstep, 0)


_seg_sum = pl.kernel(
    _seg_sum_body,
    out_type=jax.ShapeDtypeStruct((NC, N_ACC, D_FEAT), jnp.float32),
    mesh=plsc.VectorSubcoreMesh(core_axis_name="c", subcore_axis_name="s"),
    scratch_types=[
        pltpu.VMEM((CHUNKS_PER_HALF, CHUNK), jnp.int32),
        pltpu.VMEM((CHUNKS_PER_HALF, CHUNK), jnp.int32),
        pltpu.VMEM((CHUNK, D_FEAT), jnp.float32),
        pltpu.VMEM((CHUNK, D_FEAT), jnp.float32),
        pltpu.VMEM_SHARED((N_ACC, D_FEAT), jnp.float32),
        pltpu.SemaphoreType.DMA,
        pltpu.SemaphoreType.DMA,
    ],
)


def _mlp1_body(x_ref, p_ref, w1a_ref, gamma_ref, beta_ref, mean_ref, var_ref,
               w1b_ref, h_ref):
    z = x_ref[...] + p_ref[0] + p_ref[1]
    t = jnp.dot(z, w1a_ref[...], preferred_element_type=jnp.float32)
    rs = lax.rsqrt(var_ref[...] + 1e-5)
    t = (t - mean_ref[...]) * rs * gamma_ref[...] + beta_ref[...]
    t = jnp.maximum(t, 0.0)
    t = jnp.dot(t, w1b_ref[...], preferred_element_type=jnp.float32)
    h_ref[...] = jnp.maximum(t, 0.0)


def _mlp2_body(h_ref, p_ref, w2a_ref, w2b_ref, o_ref):
    z = h_ref[...] + p_ref[0] + p_ref[1]
    t = jnp.dot(z, w2a_ref[...], preferred_element_type=jnp.float32)
    t = jnp.maximum(t, 0.0)
    t = jnp.dot(t, w2b_ref[...], preferred_element_type=jnp.float32)
    o_ref[...] = jnp.maximum(t, 0.0)


M_BLK = 1000
_GRID = (N_NODES // M_BLK,)


def _row_spec(d):
    return pl.BlockSpec((M_BLK, d), lambda i: (i, 0))


def _part_spec():
    return pl.BlockSpec((2, M_BLK, D_FEAT), lambda i: (0, i, 0))


def _full_spec(a, b):
    return pl.BlockSpec((a, b), lambda i: (0, 0))


_mlp1 = pl.pallas_call(
    _mlp1_body,
    grid=_GRID,
    in_specs=[
        _row_spec(D_FEAT),
        _part_spec(),
        _full_spec(D_FEAT, HIDDEN),
        _full_spec(1, HIDDEN),
        _full_spec(1, HIDDEN),
        _full_spec(1, HIDDEN),
        _full_spec(1, HIDDEN),
        _full_spec(HIDDEN, HIDDEN),
    ],
    out_specs=_row_spec(HIDDEN),
    out_shape=jax.ShapeDtypeStruct((N_NODES, HIDDEN), jnp.float32),
)

_mlp2 = pl.pallas_call(
    _mlp2_body,
    grid=_GRID,
    in_specs=[
        _row_spec(HIDDEN),
        _part_spec(),
        _full_spec(HIDDEN, HIDDEN),
        _full_spec(HIDDEN, N_CLASSES),
    ],
    out_specs=_row_spec(N_CLASSES),
    out_shape=jax.ShapeDtypeStruct((N_NODES, N_CLASSES), jnp.float32),
)


@jax.jit
def kernel(x, edge_index, W1a, bn_gamma, bn_beta, bn_mean, bn_var, W1b, W2a, W2b):
    ei = edge_index.astype(jnp.int32)
    pad = E_PAD - N_EDGES
    pad_dst = N_NODES + (jnp.arange(pad, dtype=jnp.int32) % (N_ACC - N_NODES))
    pad_src = jnp.arange(pad, dtype=jnp.int32) % N_NODES
    src = jnp.concatenate([ei[0], pad_src])
    dst = jnp.concatenate([ei[1], pad_dst])
    src3 = src.reshape(NW, N_HALF, CHUNKS_PER_HALF, CHUNK)
    dst3 = dst.reshape(NW, N_HALF, CHUNKS_PER_HALF, CHUNK)
    zeros = jnp.zeros((N_ACC, D_FEAT), jnp.float32)

    p1 = _seg_sum(x, src3, dst3, zeros)
    h = _mlp1(x, p1, W1a,
              bn_gamma.reshape(1, HIDDEN), bn_beta.reshape(1, HIDDEN),
              bn_mean.reshape(1, HIDDEN), bn_var.reshape(1, HIDDEN), W1b)
    p2 = _seg_sum(h, src3, dst3, zeros)
    return _mlp2(h, p2, W2a, W2b)


# speedup vs baseline: 1.3045x; 1.0023x over previous
import functools

import jax
import jax.numpy as jnp
from jax import lax
from jax.experimental import pallas as pl
from jax.experimental.pallas import tpu as pltpu
from jax.experimental.pallas import tpu_sc as plsc

N_NODES = 10000
D_FEAT = 128
HIDDEN = 128
N_CLASSES = 64
N_EDGES = 320000

NC = 2
NS = 16
NW = NC * NS
CHUNK = 128
N_HALF = 2
CHUNKS_PER_HALF = 40
CHUNKS_PER_TILE = N_HALF * CHUNKS_PER_HALF
E_PAD = NW * CHUNKS_PER_TILE * CHUNK
ROWS_PER_TILE = 640
N_ACC = NS * ROWS_PER_TILE
DUMMY_ROW = N_NODES


def _seg_sum_body(table, src_idx, dst_idx, zeros, out,
                  src_v, dst_v, rows_a, rows_b, acc, sem_a, sem_b):
    c = lax.axis_index("c")
    s = lax.axis_index("s")
    wid = s * NC + c

    rows_init = 624

    @pl.when(c == 0)
    def _():
        pltpu.sync_copy(table.at[pl.ds(s * rows_init, rows_init)],
                        acc.at[pl.ds(s * rows_init, rows_init)])

    @pl.when(jnp.logical_and(c == 0, s == NS - 1))
    def _():
        rem = N_NODES - NS * rows_init
        pltpu.sync_copy(table.at[pl.ds(NS * rows_init, rem)],
                        acc.at[pl.ds(NS * rows_init, rem)])
        pltpu.sync_copy(zeros.at[pl.ds(0, N_ACC - N_NODES)],
                        acc.at[pl.ds(N_NODES, N_ACC - N_NODES)])

    @pl.when(c == 1)
    def _():
        pltpu.sync_copy(zeros.at[pl.ds(s * ROWS_PER_TILE, ROWS_PER_TILE)],
                        acc.at[pl.ds(s * ROWS_PER_TILE, ROWS_PER_TILE)])

    plsc.subcore_barrier()

    for h in range(N_HALF):
        pltpu.sync_copy(src_idx.at[wid, h], src_v)
        pltpu.sync_copy(dst_idx.at[wid, h], dst_v)
        pltpu.async_copy(table.at[src_v.at[0]], rows_a, sem_a)

        def step(g, carry):
            j = 2 * g
            pltpu.async_copy(table.at[src_v.at[j + 1]], rows_b, sem_b)
            pltpu.make_async_copy(table.at[src_v.at[j]], rows_a, sem_a).wait()
            pltpu.sync_copy(rows_a, acc.at[dst_v.at[j]], add=True)

            @pl.when(j + 2 < CHUNKS_PER_HALF)
            def _():
                pltpu.async_copy(table.at[src_v.at[j + 2]], rows_a, sem_a)

            pltpu.make_async_copy(table.at[src_v.at[j + 1]], rows_b, sem_b).wait()
            pltpu.sync_copy(rows_b, acc.at[dst_v.at[j + 1]], add=True)
            return carry

        lax.fori_loop(0, CHUNKS_PER_HALF // 2, step, 0)

    plsc.subcore_barrier()

    def wstep(k, carry):
        off = s * ROWS_PER_TILE + k * CHUNK
        pltpu.sync_copy(acc.at[pl.ds(off, CHUNK)], rows_a)
        pltpu.sync_copy(rows_a, out.at[c, pl.ds(off, CHUNK)])
        return carry

    lax.fori_loop(0, ROWS_PER_TILE // CHUNK, wstep, 0)


_seg_sum = pl.kernel(
    _seg_sum_body,
    out_type=jax.ShapeDtypeStruct((NC, N_ACC, D_FEAT), jnp.float32),
    mesh=plsc.VectorSubcoreMesh(core_axis_name="c", subcore_axis_name="s"),
    scratch_types=[
        pltpu.VMEM((CHUNKS_PER_HALF, CHUNK), jnp.int32),
        pltpu.VMEM((CHUNKS_PER_HALF, CHUNK), jnp.int32),
        pltpu.VMEM((CHUNK, D_FEAT), jnp.float32),
        pltpu.VMEM((CHUNK, D_FEAT), jnp.float32),
        pltpu.VMEM_SHARED((N_ACC, D_FEAT), jnp.float32),
        pltpu.SemaphoreType.DMA,
        pltpu.SemaphoreType.DMA,
    ],
)


def _mlp1_body(p_ref, w1a_ref, gamma_ref, beta_ref, mean_ref, var_ref,
               w1b_ref, h_ref):
    z = p_ref[0] + p_ref[1]
    t = jnp.dot(z, w1a_ref[...], preferred_element_type=jnp.float32)
    rs = lax.rsqrt(var_ref[...] + 1e-5)
    t = (t - mean_ref[...]) * rs * gamma_ref[...] + beta_ref[...]
    t = jnp.maximum(t, 0.0)
    t = jnp.dot(t, w1b_ref[...], preferred_element_type=jnp.float32)
    h_ref[...] = jnp.maximum(t, 0.0)


def _mlp2_body(p_ref, w2a_ref, w2b_ref, o_ref):
    z = p_ref[0] + p_ref[1]
    t = jnp.dot(z, w2a_ref[...], preferred_element_type=jnp.float32)
    t = jnp.maximum(t, 0.0)
    t = jnp.dot(t, w2b_ref[...], preferred_element_type=jnp.float32)
    o_ref[...] = jnp.maximum(t, 0.0)


M_BLK = 1000
_GRID = (N_NODES // M_BLK,)


def _row_spec(d):
    return pl.BlockSpec((M_BLK, d), lambda i: (i, 0))


def _part_spec():
    return pl.BlockSpec((2, M_BLK, D_FEAT), lambda i: (0, i, 0))


def _full_spec(a, b):
    return pl.BlockSpec((a, b), lambda i: (0, 0))


_mlp1 = pl.pallas_call(
    _mlp1_body,
    grid=_GRID,
    in_specs=[
        _part_spec(),
        _full_spec(D_FEAT, HIDDEN),
        _full_spec(1, HIDDEN),
        _full_spec(1, HIDDEN),
        _full_spec(1, HIDDEN),
        _full_spec(1, HIDDEN),
        _full_spec(HIDDEN, HIDDEN),
    ],
    out_specs=_row_spec(HIDDEN),
    out_shape=jax.ShapeDtypeStruct((N_NODES, HIDDEN), jnp.float32),
)

_mlp2 = pl.pallas_call(
    _mlp2_body,
    grid=_GRID,
    in_specs=[
        _part_spec(),
        _full_spec(HIDDEN, HIDDEN),
        _full_spec(HIDDEN, N_CLASSES),
    ],
    out_specs=_row_spec(N_CLASSES),
    out_shape=jax.ShapeDtypeStruct((N_NODES, N_CLASSES), jnp.float32),
)


@jax.jit
def kernel(x, edge_index, W1a, bn_gamma, bn_beta, bn_mean, bn_var, W1b, W2a, W2b):
    ei = edge_index.astype(jnp.int32)
    pad = E_PAD - N_EDGES
    pad_dst = N_NODES + (jnp.arange(pad, dtype=jnp.int32) % (N_ACC - N_NODES))
    pad_src = jnp.arange(pad, dtype=jnp.int32) % N_NODES
    src = jnp.concatenate([ei[0], pad_src])
    dst = jnp.concatenate([ei[1], pad_dst])
    src3 = src.reshape(NW, N_HALF, CHUNKS_PER_HALF, CHUNK)
    dst3 = dst.reshape(NW, N_HALF, CHUNKS_PER_HALF, CHUNK)
    zeros = jnp.zeros((N_ACC, D_FEAT), jnp.float32)

    p1 = _seg_sum(x, src3, dst3, zeros)
    h = _mlp1(p1, W1a,
              bn_gamma.reshape(1, HIDDEN), bn_beta.reshape(1, HIDDEN),
              bn_mean.reshape(1, HIDDEN), bn_var.reshape(1, HIDDEN), W1b)
    p2 = _seg_sum(h, src3, dst3, zeros)
    return _mlp2(p2, W2a, W2b)
